# Initial kernel scaffold; baseline (speedup 1.0000x reference)
import jax
import jax.numpy as jnp
from jax import lax
from jax.experimental import pallas as pl
from jax.experimental.pallas import tpu as pltpu
from jax.experimental.pallas import tpu_sc as plsc

N = 50000
D = 64
E = 800000
NG = 4
B = 64
L = 50
NC, NS = 2, 16
EROWS = 6272
EPAD = EROWS * 128
RPT = EROWS // NS
NP = 50048
STRIPE = NP // NS
SCK = 7
SCROWS = RPT // SCK
GCH = 8
NGC = SCROWS // GCH
GPT = (NG * B * L) // NS


def _i16(v):
    return jnp.full((16,), v, jnp.int32)


def _cnt_body(dst2d, zeros16, ones128, out_a, out_b, cacc, dstb):
    c = lax.axis_index("c")
    t = lax.axis_index("s")
    r0 = t * STRIPE

    def per_graph(g, _):
        pltpu.sync_copy(zeros16.at[pl.ds(r0, STRIPE)], cacc.at[pl.ds(r0, STRIPE)])
        plsc.subcore_barrier()

        def per_sck(sk, _):
            base = g * EROWS + c * (EROWS // 2) + t * (RPT // 2) + sk * 49
            pltpu.sync_copy(dst2d.at[pl.ds(base, 49)], dstb)

            def per_row(j, _):
                pltpu.sync_copy(ones128, cacc.at[dstb.at[j]], add=True)
                return 0

            lax.fori_loop(0, 49, per_row, 0)
            return 0

        lax.fori_loop(0, 4, per_sck, 0)
        plsc.subcore_barrier()

        @pl.when(c == 0)
        def _():
            pltpu.sync_copy(cacc.at[pl.ds(r0, STRIPE)], out_a.at[g, pl.ds(r0, STRIPE)])

        @pl.when(c == 1)
        def _():
            pltpu.sync_copy(cacc.at[pl.ds(r0, STRIPE)], out_b.at[g, pl.ds(r0, STRIPE)])

        plsc.subcore_barrier()
        return 0

    lax.fori_loop(0, NG, per_graph, 0)


def _seg_body(tab_l, tab_r, src2d, dst2d, w2d, zeros32,
              out_l, out_r, acc, srcb, dstb, wb, rows3, sem):
    c = lax.axis_index("c")
    t = lax.axis_index("s")
    r0 = t * STRIPE

    def per_graph(g, _):
        pltpu.sync_copy(zeros32.at[pl.ds(r0, STRIPE)], acc.at[pl.ds(r0, STRIPE)])
        plsc.subcore_barrier()

        def per_sck(sk, _):
            base = g * EROWS + t * RPT + sk * SCROWS
            pltpu.sync_copy(src2d.at[pl.ds(base, SCROWS)], srcb)
            pltpu.sync_copy(dst2d.at[pl.ds(base, SCROWS)], dstb)
            pltpu.sync_copy(w2d.at[pl.ds(base, SCROWS)], wb)

            def per_chunk(j, _):
                idx = srcb.at[pl.ds(j * GCH, GCH)]

                @pl.when(c == 0)
                def _():
                    pltpu.async_copy(tab_l.at[idx], rows3, sem).wait()

                @pl.when(c == 1)
                def _():
                    pltpu.async_copy(tab_r.at[idx], rows3, sem).wait()

                def per_row(a, _):
                    row = j * GCH + a

                    def per_e(e, _):
                        for i in range(4):
                            ee = e * 4 + i
                            wv = plsc.load_gather(wb, [_i16(row), _i16(ee)])
                            x0 = rows3[a, ee, pl.ds(0, 16)]
                            rows3[a, ee, pl.ds(0, 16)] = x0 * wv
                            x1 = rows3[a, ee, pl.ds(16, 16)]
                            rows3[a, ee, pl.ds(16, 16)] = x1 * wv
                        return 0

                    lax.fori_loop(0, 32, per_e, 0)
                    return 0

                lax.fori_loop(0, GCH, per_row, 0)
                pltpu.sync_copy(rows3, acc.at[dstb.at[pl.ds(j * GCH, GCH)]],
                                add=True)
                return 0

            lax.fori_loop(0, NGC, per_chunk, 0)
            return 0

        lax.fori_loop(0, SCK, per_sck, 0)
        plsc.subcore_barrier()

        @pl.when(c == 0)
        def _():
            pltpu.sync_copy(acc.at[pl.ds(r0, STRIPE)], out_l.at[g, pl.ds(r0, STRIPE)])

        @pl.when(c == 1)
        def _():
            pltpu.sync_copy(acc.at[pl.ds(r0, STRIPE)], out_r.at[g, pl.ds(r0, STRIPE)])

        plsc.subcore_barrier()
        return 0

    lax.fori_loop(0, NG, per_graph, 0)


def _gather_body(h1l, h1r, s2l, s2r, ca, cb, idxh, idxs,
                 ghl, ghr, gsl, gsr, gca, gcb, ib, rb32, rb16, sem):
    c = lax.axis_index("c")
    t = lax.axis_index("s")
    base = t * GPT
    nfull = GPT // 128
    tail = GPT % 128

    def gat(tab, dstbuf):
        def chunk(k, _):
            pltpu.async_copy(tab.at[ib.at[pl.ds(k * 128, 128)]],
                             dstbuf.at[pl.ds(k * 128, 128)], sem).wait()
            return 0

        lax.fori_loop(0, nfull, chunk, 0)
        pltpu.async_copy(tab.at[ib.at[pl.ds(nfull * 128, tail)]],
                         dstbuf.at[pl.ds(nfull * 128, tail)], sem).wait()

    pltpu.sync_copy(idxh.at[pl.ds(base, GPT)], ib)

    @pl.when(c == 0)
    def _():
        gat(h1l, rb32)
        pltpu.sync_copy(rb32, ghl.at[pl.ds(base, GPT)])

    @pl.when(c == 1)
    def _():
        gat(h1r, rb32)
        pltpu.sync_copy(rb32, ghr.at[pl.ds(base, GPT)])

    pltpu.sync_copy(idxs.at[pl.ds(base, GPT)], ib)

    @pl.when(c == 0)
    def _():
        gat(s2l, rb32)
        pltpu.sync_copy(rb32, gsl.at[pl.ds(base, GPT)])
        gat(ca, rb16)
        pltpu.sync_copy(rb16, gca.at[pl.ds(base, GPT)])

    @pl.when(c == 1)
    def _():
        gat(s2r, rb32)
        pltpu.sync_copy(rb32, gsr.at[pl.ds(base, GPT)])
        gat(cb, rb16)
        pltpu.sync_copy(rb16, gcb.at[pl.ds(base, GPT)])


def _dense1_body(n0l, n0r, sl, sr, ca, cb, ws, wn, bb, out_l, out_r):
    h = jnp.concatenate([n0l[0], n0r[0]], axis=-1)
    s = jnp.concatenate([sl[0], sr[0]], axis=-1)
    cnt = ca[0, :, 0] + cb[0, :, 0]
    inv = 1.0 / jnp.maximum(cnt, 1.0)
    out = (jnp.dot(h, ws[0], preferred_element_type=jnp.float32)
           + jnp.dot(s * inv[:, None], wn[0], preferred_element_type=jnp.float32)
           + bb[0][None, :])
    out = jnp.maximum(out, 0.0)
    out_l[0] = out[:, :32]
    out_r[0] = out[:, 32:]


def _attn(x, q_w, k_w, v_w):
    q = jnp.einsum('...ld,de->...le', x, q_w, preferred_element_type=jnp.float32)
    k = jnp.einsum('...ld,de->...le', x, k_w, preferred_element_type=jnp.float32)
    v = jnp.einsum('...ld,de->...le', x, v_w, preferred_element_type=jnp.float32)
    sc = jnp.einsum('...ld,...md->...lm', q, k,
                    preferred_element_type=jnp.float32) / 8.0
    w = jax.nn.softmax(sc, axis=-1)
    return jnp.einsum('...lm,...md->...ld', w, v,
                      preferred_element_type=jnp.float32)


def _final_body(ghl, ghr, gsl, gsr, gca, gcb, ws, wn, bb,
                q1, k1, v1, q2, k2, v2, out):
    gh = jnp.concatenate([ghl[...], ghr[...]], axis=-1)
    gs = jnp.concatenate([gsl[...], gsr[...]], axis=-1)
    cnt = gca[...][..., 0] + gcb[...][..., 0]
    inv = 1.0 / jnp.maximum(cnt, 1.0)
    h2 = (jnp.einsum('gbld,gde->gble', gh, ws[...],
                     preferred_element_type=jnp.float32)
          + jnp.einsum('gbld,gde->gble', gs * inv[..., None], wn[...],
                       preferred_element_type=jnp.float32)
          + bb[...][:, None, None, :])
    x = jnp.transpose(h2, (1, 0, 2, 3))
    for i in range(2):
        x = _attn(x, q1[i], k1[i], v1[i])
    x = x.sum(axis=-2)
    for i in range(2):
        x = _attn(x, q2[i], k2[i], v2[i])
    out[...] = x.sum(axis=-2)


def _sc_mesh():
    return plsc.VectorSubcoreMesh(core_axis_name="c", subcore_axis_name="s",
                                  num_cores=NC, num_subcores=NS)


def kernel(node_emb, edge_weight, sage_Wself, sage_Wneigh, sage_b,
           cQ1, cK1, cV1, cQ2, cK2, cV2,
           edge_index0, edge_index1, edge_index2, edge_index3,
           strategy_idx, batch_candidate, max_feature_length):
    f32, i32 = jnp.float32, jnp.int32

    ei = jnp.stack([edge_index0, edge_index1, edge_index2, edge_index3])
    goff = jnp.arange(NG, dtype=i32) * N
    src2d = jnp.pad(ei[:, 0, :] + goff[:, None],
                    ((0, 0), (0, EPAD - E))).reshape(NG * EROWS, 128)
    dst2d = jnp.pad(ei[:, 1, :], ((0, 0), (0, EPAD - E)),
                    constant_values=N).reshape(NG * EROWS, 128)
    w2d = jnp.pad(edge_weight.astype(f32),
                  ((0, 0), (0, EPAD - E))).reshape(NG * EROWS, 128)
    n0l = node_emb[..., :32]
    n0r = node_emb[..., 32:]
    zeros32 = jnp.zeros((NP, 32), f32)
    zeros16 = jnp.zeros((NP, 16), f32)
    ones128 = jnp.ones((128, 16), f32)
    strat = jnp.transpose(strategy_idx[batch_candidate], (1, 0, 2))
    idxh = (strat + goff[:, None, None]).reshape(-1)
    idxs = (strat + (jnp.arange(NG, dtype=i32) * NP)[:, None, None]).reshape(-1)

    cnt_call = pl.kernel(
        _cnt_body,
        out_type=[jax.ShapeDtypeStruct((NG, NP, 16), f32)] * 2,
        mesh=_sc_mesh(),
        scratch_types=[
            pltpu.VMEM_SHARED((NP, 16), f32),
            pltpu.VMEM((49, 128), i32),
        ],
    )
    cnt_a, cnt_b = cnt_call(dst2d, zeros16, ones128)

    seg_call = pl.kernel(
        _seg_body,
        out_type=[jax.ShapeDtypeStruct((NG, NP, 32), f32)] * 2,
        mesh=_sc_mesh(),
        scratch_types=[
            pltpu.VMEM_SHARED((NP, 32), f32),
            pltpu.VMEM((SCROWS, 128), i32),
            pltpu.VMEM((SCROWS, 128), i32),
            pltpu.VMEM((SCROWS, 128), f32),
            pltpu.VMEM((GCH, 128, 32), f32),
            pltpu.SemaphoreType.DMA,
        ],
    )
    s1l, s1r = seg_call(n0l.reshape(NG * N, 32), n0r.reshape(NG * N, 32),
                        src2d, dst2d, w2d, zeros32)

    rb = 2000
    spec_h = pl.BlockSpec((1, rb, 32), lambda g, i: (g, i, 0))
    spec_c = pl.BlockSpec((1, rb, 16), lambda g, i: (g, i, 0))
    spec_w = pl.BlockSpec((1, D, D), lambda g, i: (g, 0, 0))
    spec_b = pl.BlockSpec((1, D), lambda g, i: (g, 0))
    h1l, h1r = pl.pallas_call(
        _dense1_body,
        grid=(NG, N // rb),
        in_specs=[spec_h, spec_h, spec_h, spec_h, spec_c, spec_c,
                  spec_w, spec_w, spec_b],
        out_specs=[spec_h, spec_h],
        out_shape=[jax.ShapeDtypeStruct((NG, N, 32), f32)] * 2,
    )(n0l, n0r, s1l, s1r, cnt_a, cnt_b,
      sage_Wself[:, 0], sage_Wneigh[:, 0], sage_b[:, 0])

    s2l, s2r = seg_call(h1l.reshape(NG * N, 32), h1r.reshape(NG * N, 32),
                        src2d, dst2d, w2d, zeros32)

    gather_call = pl.kernel(
        _gather_body,
        out_type=[jax.ShapeDtypeStruct((NG * B * L, 32), f32)] * 4
                 + [jax.ShapeDtypeStruct((NG * B * L, 16), f32)] * 2,
        mesh=_sc_mesh(),
        scratch_types=[
            pltpu.VMEM((GPT,), i32),
            pltpu.VMEM((GPT, 32), f32),
            pltpu.VMEM((GPT, 16), f32),
            pltpu.SemaphoreType.DMA,
        ],
    )
    ghl, ghr, gsl, gsr, gca, gcb = gather_call(
        h1l.reshape(NG * N, 32), h1r.reshape(NG * N, 32),
        s2l.reshape(NG * NP, 32), s2r.reshape(NG * NP, 32),
        cnt_a.reshape(NG * NP, 16), cnt_b.reshape(NG * NP, 16),
        idxh, idxs)

    bb = 16
    spec_g32 = pl.BlockSpec((NG, bb, L, 32), lambda i: (0, i, 0, 0))
    spec_g16 = pl.BlockSpec((NG, bb, L, 16), lambda i: (0, i, 0, 0))
    spec_w4 = pl.BlockSpec((NG, D, D), lambda i: (0, 0, 0))
    spec_b4 = pl.BlockSpec((NG, D), lambda i: (0, 0))
    spec_qkv = pl.BlockSpec((2, D, D), lambda i: (0, 0, 0))
    out = pl.pallas_call(
        _final_body,
        grid=(B // bb,),
        in_specs=[spec_g32, spec_g32, spec_g32, spec_g32, spec_g16, spec_g16,
                  spec_w4, spec_w4, spec_b4,
                  spec_qkv, spec_qkv, spec_qkv, spec_qkv, spec_qkv, spec_qkv],
        out_specs=pl.BlockSpec((bb, D), lambda i: (i, 0)),
        out_shape=jax.ShapeDtypeStruct((B, D), f32),
    )(ghl.reshape(NG, B, L, 32), ghr.reshape(NG, B, L, 32),
      gsl.reshape(NG, B, L, 32), gsr.reshape(NG, B, L, 32),
      gca.reshape(NG, B, L, 16), gcb.reshape(NG, B, L, 16),
      sage_Wself[:, 1], sage_Wneigh[:, 1], sage_b[:, 1],
      cQ1, cK1, cV1, cQ2, cK2, cV2)
    return out


# speedup vs baseline: 3.8201x; 3.8201x over previous
import jax
import jax.numpy as jnp
from jax import lax
from jax.experimental import pallas as pl
from jax.experimental.pallas import tpu as pltpu
from jax.experimental.pallas import tpu_sc as plsc

N = 50000
D = 64
E = 800000
NG = 4
B = 64
L = 50
NC, NS = 2, 16
EROWS = 6272
EPAD = EROWS * 128
RPT = EROWS // NS
NP = 50048
STRIPE = NP // NS
SCK = 14
SCROWS = RPT // SCK
CHK = 136
NCHK = STRIPE // CHK
GPT = (NG * B * L) // NS


def _i16(v):
    return jnp.full((16,), v, jnp.int32)


def _zero_spmem(zeros_hbm, zbuf, spm, r0):
    pltpu.sync_copy(zeros_hbm.at[pl.ds(0, CHK)], zbuf)

    def zchunk(k, _):
        pltpu.sync_copy(zbuf, spm.at[pl.ds(r0 + k * CHK, CHK)])
        return 0

    lax.fori_loop(0, NCHK, zchunk, 0)


def _dump_spmem(spm, r0, dbuf, out_flat, o0):
    def dchunk(k, _):
        pltpu.sync_copy(spm.at[pl.ds(r0 + k * CHK, CHK)], dbuf)
        pltpu.sync_copy(dbuf, out_flat.at[pl.ds(o0 + k * CHK, CHK)])
        return 0

    lax.fori_loop(0, NCHK, dchunk, 0)


def _cnt_body(dst2d, zeros32, ones128, out_c, cacc, dstb, onesb, zbuf):
    c = lax.axis_index("c")
    t = lax.axis_index("s")
    r0 = t * STRIPE
    pltpu.sync_copy(ones128, onesb)

    def per_graph(gi, _):
        g = 2 * c + gi
        _zero_spmem(zeros32, zbuf, cacc, r0)
        plsc.subcore_barrier()

        def per_sck(sk, _):
            base = g * EROWS + t * RPT + sk * SCROWS
            pltpu.sync_copy(dst2d.at[pl.ds(base, SCROWS)], dstb)

            def per_row(j, _):
                pltpu.sync_copy(onesb, cacc.at[dstb.at[j]], add=True)
                return 0

            lax.fori_loop(0, SCROWS, per_row, 0)
            return 0

        lax.fori_loop(0, SCK, per_sck, 0)
        plsc.subcore_barrier()
        _dump_spmem(cacc, r0, zbuf, out_c, g * NP + r0)
        plsc.subcore_barrier()
        return 0

    lax.fori_loop(0, 2, per_graph, 0)


def _seg_body(tab_l, tab_r, src2d, dst2d, w2d, zeros32,
              out_l, out_r, acc, srcb, dstb, wb, rowsf, zbuf, sem):
    c = lax.axis_index("c")
    t = lax.axis_index("s")
    r0 = t * STRIPE

    def per_graph(g, _):
        _zero_spmem(zeros32, zbuf, acc, r0)
        plsc.subcore_barrier()

        def per_sck(sk, _):
            base = g * EROWS + t * RPT + sk * SCROWS
            pltpu.sync_copy(src2d.at[pl.ds(base, SCROWS)], srcb)
            pltpu.sync_copy(dst2d.at[pl.ds(base, SCROWS)], dstb)
            pltpu.sync_copy(w2d.at[pl.ds(base, SCROWS)], wb)

            def per_chunk(j, _):
                idx = srcb.at[j]

                @pl.when(c == 0)
                def _():
                    pltpu.async_copy(tab_l.at[idx], rowsf, sem).wait()

                @pl.when(c == 1)
                def _():
                    pltpu.async_copy(tab_r.at[idx], rowsf, sem).wait()

                def per_e(e, _):
                    for i in range(4):
                        ee = e * 4 + i
                        wv = plsc.load_gather(wb, [_i16(j), _i16(ee)])
                        x0 = rowsf[ee, pl.ds(0, 16)]
                        rowsf[ee, pl.ds(0, 16)] = x0 * wv
                        x1 = rowsf[ee, pl.ds(16, 16)]
                        rowsf[ee, pl.ds(16, 16)] = x1 * wv
                    return 0

                lax.fori_loop(0, 32, per_e, 0)
                pltpu.sync_copy(rowsf, acc.at[dstb.at[j]], add=True)
                return 0

            lax.fori_loop(0, SCROWS, per_chunk, 0)
            return 0

        lax.fori_loop(0, SCK, per_sck, 0)
        plsc.subcore_barrier()

        @pl.when(c == 0)
        def _():
            _dump_spmem(acc, r0, zbuf, out_l, g * NP + r0)

        @pl.when(c == 1)
        def _():
            _dump_spmem(acc, r0, zbuf, out_r, g * NP + r0)

        plsc.subcore_barrier()
        return 0

    lax.fori_loop(0, NG, per_graph, 0)


def _gather_body(h1l, h1r, s2l, s2r, ca, idxh, idxs,
                 ghl, ghr, gsl, gsr, gca, ib, rbuf, sem):
    c = lax.axis_index("c")
    t = lax.axis_index("s")
    base = t * GPT
    nfull = GPT // 128
    tail = GPT % 128

    def gat(tab):
        def chunk(k, _):
            pltpu.async_copy(tab.at[ib.at[pl.ds(k * 128, 128)]],
                             rbuf.at[pl.ds(k * 128, 128)], sem).wait()
            return 0

        lax.fori_loop(0, nfull, chunk, 0)
        pltpu.async_copy(tab.at[ib.at[pl.ds(nfull * 128, tail)]],
                         rbuf.at[pl.ds(nfull * 128, tail)], sem).wait()

    pltpu.sync_copy(idxh.at[pl.ds(base, GPT)], ib)

    @pl.when(c == 0)
    def _():
        gat(h1l)
        pltpu.sync_copy(rbuf, ghl.at[pl.ds(base, GPT)])

    @pl.when(c == 1)
    def _():
        gat(h1r)
        pltpu.sync_copy(rbuf, ghr.at[pl.ds(base, GPT)])

    pltpu.sync_copy(idxs.at[pl.ds(base, GPT)], ib)

    @pl.when(c == 0)
    def _():
        gat(s2l)
        pltpu.sync_copy(rbuf, gsl.at[pl.ds(base, GPT)])
        gat(ca)
        pltpu.sync_copy(rbuf, gca.at[pl.ds(base, GPT)])

    @pl.when(c == 1)
    def _():
        gat(s2r)
        pltpu.sync_copy(rbuf, gsr.at[pl.ds(base, GPT)])


def _dense1_body(n0l, n0r, sl, sr, ca, ws, wn, bb, out_l, out_r):
    h = jnp.concatenate([n0l[0], n0r[0]], axis=-1)
    s = jnp.concatenate([sl[0], sr[0]], axis=-1)
    cnt = ca[0, :, 0]
    inv = 1.0 / jnp.maximum(cnt, 1.0)
    out = (jnp.dot(h, ws[0], preferred_element_type=jnp.float32)
           + jnp.dot(s * inv[:, None], wn[0], preferred_element_type=jnp.float32)
           + bb[0, 0][None, :])
    out = jnp.maximum(out, 0.0)
    out_l[0] = out[:, :32]
    out_r[0] = out[:, 32:]


def _attn(x, q_w, k_w, v_w):
    q = jnp.einsum('...ld,de->...le', x, q_w, preferred_element_type=jnp.float32)
    k = jnp.einsum('...ld,de->...le', x, k_w, preferred_element_type=jnp.float32)
    v = jnp.einsum('...ld,de->...le', x, v_w, preferred_element_type=jnp.float32)
    sc = jnp.einsum('...ld,...md->...lm', q, k,
                    preferred_element_type=jnp.float32) / 8.0
    w = jax.nn.softmax(sc, axis=-1)
    return jnp.einsum('...lm,...md->...ld', w, v,
                      preferred_element_type=jnp.float32)


def _final_body(ghl, ghr, gsl, gsr, gca, ws, wn, bb,
                q1, k1, v1, q2, k2, v2, out):
    gh = jnp.concatenate([ghl[...], ghr[...]], axis=-1)
    gs = jnp.concatenate([gsl[...], gsr[...]], axis=-1)
    cnt = gca[...][..., 0]
    inv = 1.0 / jnp.maximum(cnt, 1.0)
    h2 = (jnp.einsum('gbld,gde->gble', gh, ws[...],
                     preferred_element_type=jnp.float32)
          + jnp.einsum('gbld,gde->gble', gs * inv[..., None], wn[...],
                       preferred_element_type=jnp.float32)
          + bb[...][:, None, None, :])
    x = jnp.transpose(h2, (1, 0, 2, 3))
    nbb = x.shape[0]
    x = x.reshape(nbb * 4, L, D)
    for i in range(2):
        x = _attn(x, q1[i], k1[i], v1[i])
    x = x.sum(axis=-2).reshape(nbb, 4, D)
    for i in range(2):
        x = _attn(x, q2[i], k2[i], v2[i])
    out[...] = x.sum(axis=-2)


def _sc_mesh():
    return plsc.VectorSubcoreMesh(core_axis_name="c", subcore_axis_name="s",
                                  num_cores=NC, num_subcores=NS)


def kernel(node_emb, edge_weight, sage_Wself, sage_Wneigh, sage_b,
           cQ1, cK1, cV1, cQ2, cK2, cV2,
           edge_index0, edge_index1, edge_index2, edge_index3,
           strategy_idx, batch_candidate, max_feature_length):
    f32, i32 = jnp.float32, jnp.int32

    ei = jnp.stack([edge_index0, edge_index1, edge_index2, edge_index3])
    goff = jnp.arange(NG, dtype=i32) * N
    src2d = jnp.pad(ei[:, 0, :] + goff[:, None],
                    ((0, 0), (0, EPAD - E))).reshape(NG * EROWS, 128)
    dst2d = jnp.pad(ei[:, 1, :], ((0, 0), (0, EPAD - E)),
                    constant_values=N).reshape(NG * EROWS, 128)
    w2d = jnp.pad(edge_weight.astype(f32),
                  ((0, 0), (0, EPAD - E))).reshape(NG * EROWS, 128)
    n0l = node_emb[..., :32]
    n0r = node_emb[..., 32:]
    zeros32 = jnp.zeros((NP, 32), f32)
    ones128 = jnp.ones((128, 32), f32)
    strat = jnp.transpose(strategy_idx[batch_candidate], (1, 0, 2))
    idxh = (strat + goff[:, None, None]).reshape(-1)
    idxs = (strat + (jnp.arange(NG, dtype=i32) * NP)[:, None, None]).reshape(-1)

    cnt_call = pl.kernel(
        _cnt_body,
        out_type=jax.ShapeDtypeStruct((NG * NP, 32), f32),
        mesh=_sc_mesh(),
        scratch_types=[
            pltpu.VMEM_SHARED((NP, 32), f32),
            pltpu.VMEM((SCROWS, 128), i32),
            pltpu.VMEM((128, 32), f32),
            pltpu.VMEM((CHK, 32), f32),
        ],
        compiler_params=pltpu.CompilerParams(use_tc_tiling_on_sc=False),
    )
    cnt_a = cnt_call(dst2d, zeros32, ones128).reshape(NG, NP, 32)

    seg_call = pl.kernel(
        _seg_body,
        out_type=[jax.ShapeDtypeStruct((NG * NP, 32), f32)] * 2,
        mesh=_sc_mesh(),
        scratch_types=[
            pltpu.VMEM_SHARED((NP, 32), f32),
            pltpu.VMEM((SCROWS, 128), i32),
            pltpu.VMEM((SCROWS, 128), i32),
            pltpu.VMEM((SCROWS, 128), f32),
            pltpu.VMEM((128, 32), f32),
            pltpu.VMEM((CHK, 32), f32),
            pltpu.SemaphoreType.DMA,
        ],
        compiler_params=pltpu.CompilerParams(needs_layout_passes=False,
                                             use_tc_tiling_on_sc=False),
    )
    s1l, s1r = seg_call(n0l.reshape(NG * N, 32), n0r.reshape(NG * N, 32),
                        src2d, dst2d, w2d, zeros32)
    s1l = s1l.reshape(NG, NP, 32)
    s1r = s1r.reshape(NG, NP, 32)

    rb = 2000
    spec_h = pl.BlockSpec((1, rb, 32), lambda g, i: (g, i, 0))
    spec_c = pl.BlockSpec((1, rb, 32), lambda g, i: (g, i, 0))
    spec_w = pl.BlockSpec((1, D, D), lambda g, i: (g, 0, 0))
    spec_b = pl.BlockSpec((1, 1, D), lambda g, i: (g, 0, 0))
    h1l, h1r = pl.pallas_call(
        _dense1_body,
        grid=(NG, N // rb),
        in_specs=[spec_h, spec_h, spec_h, spec_h, spec_c,
                  spec_w, spec_w, spec_b],
        out_specs=[spec_h, spec_h],
        out_shape=[jax.ShapeDtypeStruct((NG, N, 32), f32)] * 2,
    )(n0l, n0r, s1l, s1r, cnt_a,
      sage_Wself[:, 0], sage_Wneigh[:, 0], sage_b[:, 0].reshape(NG, 1, D))

    s2l, s2r = seg_call(h1l.reshape(NG * N, 32), h1r.reshape(NG * N, 32),
                        src2d, dst2d, w2d, zeros32)

    gather_call = pl.kernel(
        _gather_body,
        out_type=[jax.ShapeDtypeStruct((NG * B * L, 32), f32)] * 5,
        mesh=_sc_mesh(),
        scratch_types=[
            pltpu.VMEM((GPT,), i32),
            pltpu.VMEM((GPT, 32), f32),
            pltpu.SemaphoreType.DMA,
        ],
        compiler_params=pltpu.CompilerParams(use_tc_tiling_on_sc=False),
    )
    ghl, ghr, gsl, gsr, gca = gather_call(
        h1l.reshape(NG * N, 32), h1r.reshape(NG * N, 32),
        s2l, s2r, cnt_a.reshape(NG * NP, 32),
        idxh, idxs)

    bb = 16
    spec_g32 = pl.BlockSpec((NG, bb, L, 32), lambda i: (0, i, 0, 0))
    spec_w4 = pl.BlockSpec((NG, D, D), lambda i: (0, 0, 0))
    spec_b4 = pl.BlockSpec((NG, D), lambda i: (0, 0))
    spec_qkv = pl.BlockSpec((2, D, D), lambda i: (0, 0, 0))
    out = pl.pallas_call(
        _final_body,
        grid=(B // bb,),
        in_specs=[spec_g32, spec_g32, spec_g32, spec_g32, spec_g32,
                  spec_w4, spec_w4, spec_b4,
                  spec_qkv, spec_qkv, spec_qkv, spec_qkv, spec_qkv, spec_qkv],
        out_specs=pl.BlockSpec((bb, D), lambda i: (i, 0)),
        out_shape=jax.ShapeDtypeStruct((B, D), f32),
    )(ghl.reshape(NG, B, L, 32), ghr.reshape(NG, B, L, 32),
      gsl.reshape(NG, B, L, 32), gsr.reshape(NG, B, L, 32),
      gca.reshape(NG, B, L, 32),
      sage_Wself[:, 1], sage_Wneigh[:, 1], sage_b[:, 1],
      cQ1, cK1, cV1, cQ2, cK2, cV2)
    return out


# speedup vs baseline: 4.4676x; 1.1695x over previous
import jax
import jax.numpy as jnp
from jax import lax
from jax.experimental import pallas as pl
from jax.experimental.pallas import tpu as pltpu
from jax.experimental.pallas import tpu_sc as plsc

N = 50000
D = 64
E = 800000
NG = 4
B = 64
L = 50
NC, NS = 2, 16
EROWS = 6272
EPAD = EROWS * 128
RPT = EROWS // NS
NP = 50176
STRIPE = NP // NS
SCK = 28
SCROWS = RPT // SCK
CHK = 112
NCHK = STRIPE // CHK
GPT = (NG * B * L) // NS


def _i16(v):
    return jnp.full((16,), v, jnp.int32)


def _zero_spmem(zeros_hbm, zbuf, spm, r0):
    pltpu.sync_copy(zeros_hbm.at[pl.ds(0, CHK)], zbuf)

    def zchunk(k, _):
        pltpu.sync_copy(zbuf, spm.at[pl.ds(r0 + k * CHK, CHK)])
        return 0

    lax.fori_loop(0, NCHK, zchunk, 0)


def _dump_spmem(spm, r0, dbuf, out_flat, o0):
    def dchunk(k, _):
        pltpu.sync_copy(spm.at[pl.ds(r0 + k * CHK, CHK)], dbuf)
        pltpu.sync_copy(dbuf, out_flat.at[pl.ds(o0 + k * CHK, CHK)])
        return 0

    lax.fori_loop(0, NCHK, dchunk, 0)


def _cnt_body(dst2d, zeros32, ones128, out_c, cacc, dstb, onesb, zbuf):
    c = lax.axis_index("c")
    t = lax.axis_index("s")
    r0 = t * STRIPE
    pltpu.sync_copy(ones128, onesb)

    def per_graph(gi, _):
        g = 2 * c + gi
        _zero_spmem(zeros32, zbuf, cacc, r0)
        plsc.subcore_barrier()

        def per_sck(sk, _):
            base = g * EROWS + t * RPT + sk * SCROWS
            pltpu.sync_copy(dst2d.at[pl.ds(base, SCROWS)], dstb)

            def per_row(j, _):
                pltpu.sync_copy(onesb, cacc.at[dstb.at[j]], add=True)
                return 0

            lax.fori_loop(0, SCROWS, per_row, 0)
            return 0

        lax.fori_loop(0, SCK, per_sck, 0)
        plsc.subcore_barrier()
        _dump_spmem(cacc, r0, zbuf, out_c, g * NP + r0)
        plsc.subcore_barrier()
        return 0

    lax.fori_loop(0, 2, per_graph, 0)


def _seg_body(tab_l, tab_r, src2d, dst2d, w2d, zeros32,
              out_l, out_r, acc, srcb, dstb, wb, g0, g1, s0, s1, zbuf,
              gsem, ssem):
    c = lax.axis_index("c")
    t = lax.axis_index("s")
    r0 = t * STRIPE
    gbuf = (g0, g1)
    sbuf = (s0, s1)

    def fire_gather(j, buf):
        idx = srcb.at[j]

        @pl.when(c == 0)
        def _():
            pltpu.async_copy(tab_l.at[idx], buf, gsem)

        @pl.when(c == 1)
        def _():
            pltpu.async_copy(tab_r.at[idx], buf, gsem)

    def wait_gather():
        pltpu.make_async_copy(tab_l.at[srcb.at[0]], g0, gsem).wait()

    def wait_scatter():
        pltpu.make_async_copy(s0, acc.at[dstb.at[0]], ssem).wait()

    def per_graph(g, _):
        _zero_spmem(zeros32, zbuf, acc, r0)
        plsc.subcore_barrier()

        def per_sck(sk, _):
            base = g * EROWS + t * RPT + sk * SCROWS
            pltpu.sync_copy(src2d.at[pl.ds(base, SCROWS)], srcb)
            pltpu.sync_copy(dst2d.at[pl.ds(base, SCROWS)], dstb)
            pltpu.sync_copy(w2d.at[pl.ds(base, SCROWS)], wb)
            fire_gather(0, g0)
            fire_gather(1, g1)

            def per_step(k, _):
                for b in range(2):
                    j = k * 2 + b
                    wait_gather()

                    @pl.when(j >= 2)
                    def _():
                        wait_scatter()

                    gb = gbuf[b]
                    sb = sbuf[b]

                    def per_e(e, _):
                        for i in range(4):
                            ee = e * 4 + i
                            wv = plsc.load_gather(wb, [_i16(j), _i16(ee)])
                            sb[ee, pl.ds(0, 16)] = gb[ee, pl.ds(0, 16)] * wv
                            sb[ee, pl.ds(16, 16)] = gb[ee, pl.ds(16, 16)] * wv
                        return 0

                    lax.fori_loop(0, 32, per_e, 0)
                    pltpu.async_copy(sb, acc.at[dstb.at[j]], ssem, add=True)

                    @pl.when(j + 2 < SCROWS)
                    def _():
                        fire_gather(j + 2, gb)

                return 0

            lax.fori_loop(0, SCROWS // 2, per_step, 0)
            wait_scatter()
            wait_scatter()
            return 0

        lax.fori_loop(0, SCK, per_sck, 0)
        plsc.subcore_barrier()

        @pl.when(c == 0)
        def _():
            _dump_spmem(acc, r0, zbuf, out_l, g * NP + r0)

        @pl.when(c == 1)
        def _():
            _dump_spmem(acc, r0, zbuf, out_r, g * NP + r0)

        plsc.subcore_barrier()
        return 0

    lax.fori_loop(0, NG, per_graph, 0)


def _gather_body(h1l, h1r, s2l, s2r, ca, idxh, idxs,
                 ghl, ghr, gsl, gsr, gca, ib, rbuf, sem):
    c = lax.axis_index("c")
    t = lax.axis_index("s")
    base = t * GPT
    nfull = GPT // 128
    tail = GPT % 128

    def gat(tab):
        def chunk(k, _):
            pltpu.async_copy(tab.at[ib.at[pl.ds(k * 128, 128)]],
                             rbuf.at[pl.ds(k * 128, 128)], sem).wait()
            return 0

        lax.fori_loop(0, nfull, chunk, 0)
        pltpu.async_copy(tab.at[ib.at[pl.ds(nfull * 128, tail)]],
                         rbuf.at[pl.ds(nfull * 128, tail)], sem).wait()

    pltpu.sync_copy(idxh.at[pl.ds(base, GPT)], ib)

    @pl.when(c == 0)
    def _():
        gat(h1l)
        pltpu.sync_copy(rbuf, ghl.at[pl.ds(base, GPT)])

    @pl.when(c == 1)
    def _():
        gat(h1r)
        pltpu.sync_copy(rbuf, ghr.at[pl.ds(base, GPT)])

    pltpu.sync_copy(idxs.at[pl.ds(base, GPT)], ib)

    @pl.when(c == 0)
    def _():
        gat(s2l)
        pltpu.sync_copy(rbuf, gsl.at[pl.ds(base, GPT)])
        gat(ca)
        pltpu.sync_copy(rbuf, gca.at[pl.ds(base, GPT)])

    @pl.when(c == 1)
    def _():
        gat(s2r)
        pltpu.sync_copy(rbuf, gsr.at[pl.ds(base, GPT)])


def _dense1_body(n0l, n0r, sl, sr, ca, ws, wn, bb, out_l, out_r):
    h = jnp.concatenate([n0l[0], n0r[0]], axis=-1)
    s = jnp.concatenate([sl[0], sr[0]], axis=-1)
    cnt = ca[0, :, 0]
    inv = 1.0 / jnp.maximum(cnt, 1.0)
    out = (jnp.dot(h, ws[0], preferred_element_type=jnp.float32)
           + jnp.dot(s * inv[:, None], wn[0], preferred_element_type=jnp.float32)
           + bb[0, 0][None, :])
    out = jnp.maximum(out, 0.0)
    out_l[0] = out[:, :32]
    out_r[0] = out[:, 32:]


def _attn(x, q_w, k_w, v_w):
    q = jnp.einsum('...ld,de->...le', x, q_w, preferred_element_type=jnp.float32)
    k = jnp.einsum('...ld,de->...le', x, k_w, preferred_element_type=jnp.float32)
    v = jnp.einsum('...ld,de->...le', x, v_w, preferred_element_type=jnp.float32)
    sc = jnp.einsum('...ld,...md->...lm', q, k,
                    preferred_element_type=jnp.float32) / 8.0
    w = jax.nn.softmax(sc, axis=-1)
    return jnp.einsum('...lm,...md->...ld', w, v,
                      preferred_element_type=jnp.float32)


def _final_body(ghl, ghr, gsl, gsr, gca, ws, wn, bb,
                q1, k1, v1, q2, k2, v2, out):
    gh = jnp.concatenate([ghl[...], ghr[...]], axis=-1)
    gs = jnp.concatenate([gsl[...], gsr[...]], axis=-1)
    cnt = gca[...][..., 0]
    inv = 1.0 / jnp.maximum(cnt, 1.0)
    h2 = (jnp.einsum('gbld,gde->gble', gh, ws[...],
                     preferred_element_type=jnp.float32)
          + jnp.einsum('gbld,gde->gble', gs * inv[..., None], wn[...],
                       preferred_element_type=jnp.float32)
          + bb[...][:, None, None, :])
    x = jnp.transpose(h2, (1, 0, 2, 3))
    nbb = x.shape[0]
    x = x.reshape(nbb * 4, L, D)
    for i in range(2):
        x = _attn(x, q1[i], k1[i], v1[i])
    x = x.sum(axis=-2).reshape(nbb, 4, D)
    for i in range(2):
        x = _attn(x, q2[i], k2[i], v2[i])
    out[...] = x.sum(axis=-2)


def _sc_mesh():
    return plsc.VectorSubcoreMesh(core_axis_name="c", subcore_axis_name="s",
                                  num_cores=NC, num_subcores=NS)


def kernel(node_emb, edge_weight, sage_Wself, sage_Wneigh, sage_b,
           cQ1, cK1, cV1, cQ2, cK2, cV2,
           edge_index0, edge_index1, edge_index2, edge_index3,
           strategy_idx, batch_candidate, max_feature_length):
    f32, i32 = jnp.float32, jnp.int32

    ei = jnp.stack([edge_index0, edge_index1, edge_index2, edge_index3])
    goff = jnp.arange(NG, dtype=i32) * N
    src2d = jnp.pad(ei[:, 0, :] + goff[:, None],
                    ((0, 0), (0, EPAD - E))).reshape(NG * EROWS, 128)
    dst2d = jnp.pad(ei[:, 1, :], ((0, 0), (0, EPAD - E)),
                    constant_values=N).reshape(NG * EROWS, 128)
    w2d = jnp.pad(edge_weight.astype(f32),
                  ((0, 0), (0, EPAD - E))).reshape(NG * EROWS, 128)
    n0l = node_emb[..., :32]
    n0r = node_emb[..., 32:]
    zeros32 = jnp.zeros((NP, 32), f32)
    ones128 = jnp.ones((128, 32), f32)
    strat = jnp.transpose(strategy_idx[batch_candidate], (1, 0, 2))
    idxh = (strat + goff[:, None, None]).reshape(-1)
    idxs = (strat + (jnp.arange(NG, dtype=i32) * NP)[:, None, None]).reshape(-1)

    cnt_call = pl.kernel(
        _cnt_body,
        out_type=jax.ShapeDtypeStruct((NG * NP, 32), f32),
        mesh=_sc_mesh(),
        scratch_types=[
            pltpu.VMEM_SHARED((NP, 32), f32),
            pltpu.VMEM((SCROWS, 128), i32),
            pltpu.VMEM((128, 32), f32),
            pltpu.VMEM((CHK, 32), f32),
        ],
        compiler_params=pltpu.CompilerParams(use_tc_tiling_on_sc=False),
    )
    cnt_a = cnt_call(dst2d, zeros32, ones128).reshape(NG, NP, 32)

    seg_call = pl.kernel(
        _seg_body,
        out_type=[jax.ShapeDtypeStruct((NG * NP, 32), f32)] * 2,
        mesh=_sc_mesh(),
        scratch_types=[
            pltpu.VMEM_SHARED((NP, 32), f32),
            pltpu.VMEM((SCROWS, 128), i32),
            pltpu.VMEM((SCROWS, 128), i32),
            pltpu.VMEM((SCROWS, 128), f32),
            pltpu.VMEM((128, 32), f32),
            pltpu.VMEM((128, 32), f32),
            pltpu.VMEM((128, 32), f32),
            pltpu.VMEM((128, 32), f32),
            pltpu.VMEM((CHK, 32), f32),
            pltpu.SemaphoreType.DMA,
            pltpu.SemaphoreType.DMA,
        ],
        compiler_params=pltpu.CompilerParams(needs_layout_passes=False,
                                             use_tc_tiling_on_sc=False),
    )
    s1l, s1r = seg_call(n0l.reshape(NG * N, 32), n0r.reshape(NG * N, 32),
                        src2d, dst2d, w2d, zeros32)
    s1l = s1l.reshape(NG, NP, 32)
    s1r = s1r.reshape(NG, NP, 32)

    rb = 2000
    spec_h = pl.BlockSpec((1, rb, 32), lambda g, i: (g, i, 0))
    spec_c = pl.BlockSpec((1, rb, 32), lambda g, i: (g, i, 0))
    spec_w = pl.BlockSpec((1, D, D), lambda g, i: (g, 0, 0))
    spec_b = pl.BlockSpec((1, 1, D), lambda g, i: (g, 0, 0))
    h1l, h1r = pl.pallas_call(
        _dense1_body,
        grid=(NG, N // rb),
        in_specs=[spec_h, spec_h, spec_h, spec_h, spec_c,
                  spec_w, spec_w, spec_b],
        out_specs=[spec_h, spec_h],
        out_shape=[jax.ShapeDtypeStruct((NG, N, 32), f32)] * 2,
    )(n0l, n0r, s1l, s1r, cnt_a,
      sage_Wself[:, 0], sage_Wneigh[:, 0], sage_b[:, 0].reshape(NG, 1, D))

    s2l, s2r = seg_call(h1l.reshape(NG * N, 32), h1r.reshape(NG * N, 32),
                        src2d, dst2d, w2d, zeros32)

    gather_call = pl.kernel(
        _gather_body,
        out_type=[jax.ShapeDtypeStruct((NG * B * L, 32), f32)] * 5,
        mesh=_sc_mesh(),
        scratch_types=[
            pltpu.VMEM((GPT,), i32),
            pltpu.VMEM((GPT, 32), f32),
            pltpu.SemaphoreType.DMA,
        ],
        compiler_params=pltpu.CompilerParams(use_tc_tiling_on_sc=False),
    )
    ghl, ghr, gsl, gsr, gca = gather_call(
        h1l.reshape(NG * N, 32), h1r.reshape(NG * N, 32),
        s2l, s2r, cnt_a.reshape(NG * NP, 32),
        idxh, idxs)

    bb = 16
    spec_g32 = pl.BlockSpec((NG, bb, L, 32), lambda i: (0, i, 0, 0))
    spec_w4 = pl.BlockSpec((NG, D, D), lambda i: (0, 0, 0))
    spec_b4 = pl.BlockSpec((NG, D), lambda i: (0, 0))
    spec_qkv = pl.BlockSpec((2, D, D), lambda i: (0, 0, 0))
    out = pl.pallas_call(
        _final_body,
        grid=(B // bb,),
        in_specs=[spec_g32, spec_g32, spec_g32, spec_g32, spec_g32,
                  spec_w4, spec_w4, spec_b4,
                  spec_qkv, spec_qkv, spec_qkv, spec_qkv, spec_qkv, spec_qkv],
        out_specs=pl.BlockSpec((bb, D), lambda i: (i, 0)),
        out_shape=jax.ShapeDtypeStruct((B, D), f32),
    )(ghl.reshape(NG, B, L, 32), ghr.reshape(NG, B, L, 32),
      gsl.reshape(NG, B, L, 32), gsr.reshape(NG, B, L, 32),
      gca.reshape(NG, B, L, 32),
      sage_Wself[:, 1], sage_Wneigh[:, 1], sage_b[:, 1],
      cQ1, cK1, cV1, cQ2, cK2, cV2)
    return out
